# baseline (speedup 1.0000x reference)
import functools

import jax
import jax.numpy as jnp
from jax import lax
from jax.experimental import pallas as pl
from jax.experimental.pallas import tpu as pltpu
from jax.experimental.pallas import tpu_sc as plsc

B = 1024
N = 4096
M = 16384
E = 16384
L = 16
EG = E // L
MG = M // L


CH = 4096


def _sc_body(x_hbm, wi_hbm, val_hbm, out_hbm,
             pk_v, vals_v,
             acc00, acc01, acc10, acc11,
             xb00, xb01, xb10, xb11,
             tmp_r, tmp_c,
             xsem0, xsem1, osem0, osem1):
    info = plsc.get_sparse_core_info()
    nc = info.num_cores
    per_tile = B // (nc * info.num_subcores)
    wid = lax.axis_index("s") * nc + lax.axis_index("c")
    b_base = wid * per_tile

    pltpu.sync_copy(val_hbm, vals_v)
    for ch in range(E // CH):
        pltpu.sync_copy(wi_hbm.at[0, pl.ds(ch * CH, CH)], tmp_r)
        pltpu.sync_copy(wi_hbm.at[1, pl.ds(ch * CH, CH)], tmp_c)

        @plsc.parallel_loop(0, CH // L, unroll=8)
        def _pack(k):
            s = pl.ds(k * L, L)
            pk_v[pl.ds(ch * CH + k * L, L)] = (tmp_r[s] << 12) | tmp_c[s]

    accs = ((acc00, acc01), (acc10, acc11))
    xbs = ((xb00, xb01), (xb10, xb11))
    xsems = (xsem0, xsem1)
    osems = (osem0, osem1)
    n_pairs = per_tile // 2

    pltpu.async_copy(x_hbm.at[b_base], xb00, xsem0)
    pltpu.async_copy(x_hbm.at[b_base + 1], xb01, xsem0)

    @plsc.parallel_loop(0, MG, unroll=8)
    def _zero0(k):
        s = pl.ds(k * L, L)
        z = jnp.zeros((L,), jnp.float32)
        acc00[s] = z
        acc01[s] = z

    HALF = EG // 2

    for p in range(n_pairs):
        par = p & 1
        a0, a1 = accs[par]
        o0, o1 = accs[1 - par]
        x0, x1 = xbs[par]
        b0 = b_base + 2 * p

        if p + 1 < n_pairs:
            nxt = b_base + 2 * (p + 1)
            pltpu.async_copy(x_hbm.at[nxt], xbs[1 - par][0], xsems[1 - par])
            pltpu.async_copy(x_hbm.at[nxt + 1], xbs[1 - par][1], xsems[1 - par])

        pltpu.make_async_copy(x_hbm.at[b0], x0, xsems[par]).wait()
        pltpu.make_async_copy(x_hbm.at[b0 + 1], x1, xsems[par]).wait()

        @plsc.parallel_loop(0, HALF, unroll=4)
        def _entry_a(g):
            s = pl.ds(g * L, L)
            pk = pk_v[s]
            v = vals_v[s]
            r = pk >> 12
            c = pk & 4095
            plsc.addupdate_scatter(a0, [r], plsc.load_gather(x0, [c]) * v)
            plsc.addupdate_scatter(a1, [r], plsc.load_gather(x1, [c]) * v)

        if p >= 1:
            prv = b_base + 2 * (p - 1)
            pltpu.make_async_copy(o0, out_hbm.at[prv], osems[1 - par]).wait()
            pltpu.make_async_copy(o1, out_hbm.at[prv + 1], osems[1 - par]).wait()

        @plsc.parallel_loop(HALF, EG, unroll=4)
        def _entry_b(g):
            s = pl.ds(g * L, L)
            pk = pk_v[s]
            v = vals_v[s]
            r = pk >> 12
            c = pk & 4095
            plsc.addupdate_scatter(a0, [r], plsc.load_gather(x0, [c]) * v)
            plsc.addupdate_scatter(a1, [r], plsc.load_gather(x1, [c]) * v)
            z = jnp.zeros((L,), jnp.float32)
            k = (g - HALF) * 2
            o0[pl.ds(k * L, L)] = z
            o0[pl.ds((k + 1) * L, L)] = z
            o1[pl.ds(k * L, L)] = z
            o1[pl.ds((k + 1) * L, L)] = z

        pltpu.async_copy(a0, out_hbm.at[b0], osems[par])
        pltpu.async_copy(a1, out_hbm.at[b0 + 1], osems[par])

    p = n_pairs - 1
    par = p & 1
    a0, a1 = accs[par]
    b0 = b_base + 2 * p
    pltpu.make_async_copy(a0, out_hbm.at[b0], osems[par]).wait()
    pltpu.make_async_copy(a1, out_hbm.at[b0 + 1], osems[par]).wait()


@jax.jit
def _sc_spmm(x, wi, vals):
    mesh = plsc.VectorSubcoreMesh(core_axis_name="c", subcore_axis_name="s")
    kfn = functools.partial(
        pl.kernel,
        out_type=jax.ShapeDtypeStruct((B, M), jnp.float32),
        mesh=mesh,
        compiler_params=pltpu.CompilerParams(needs_layout_passes=False),
        scratch_types=[
            pltpu.VMEM((E,), jnp.int32),
            pltpu.VMEM((E,), jnp.float32),
            pltpu.VMEM((M,), jnp.float32),
            pltpu.VMEM((M,), jnp.float32),
            pltpu.VMEM((M,), jnp.float32),
            pltpu.VMEM((M,), jnp.float32),
            pltpu.VMEM((N,), jnp.float32),
            pltpu.VMEM((N,), jnp.float32),
            pltpu.VMEM((N,), jnp.float32),
            pltpu.VMEM((N,), jnp.float32),
            pltpu.VMEM((CH,), jnp.int32),
            pltpu.VMEM((CH,), jnp.int32),
            pltpu.SemaphoreType.DMA,
            pltpu.SemaphoreType.DMA,
            pltpu.SemaphoreType.DMA,
            pltpu.SemaphoreType.DMA,
        ],
    )(_sc_body)
    return kfn(x, wi, vals)


def kernel(input, weight_indices, weight_values):
    return _sc_spmm(input, weight_indices.astype(jnp.int32), weight_values)


# speedup vs baseline: 1.0643x; 1.0643x over previous
import functools

import jax
import jax.numpy as jnp
from jax import lax
from jax.experimental import pallas as pl
from jax.experimental.pallas import tpu as pltpu
from jax.experimental.pallas import tpu_sc as plsc

B = 1024
N = 4096
M = 16384
E = 16384
L = 16
EG = E // L
MG = M // L


def _sc_body(x_hbm, pk_hbm, val_hbm, out_hbm,
             pk_v, vals_v,
             acc00, acc01, acc10, acc11,
             xb00, xb01, xb10, xb11,
             xsem0, xsem1, osem0, osem1):
    info = plsc.get_sparse_core_info()
    nc = info.num_cores
    per_tile = B // (nc * info.num_subcores)
    wid = lax.axis_index("s") * nc + lax.axis_index("c")
    b_base = wid * per_tile

    pltpu.sync_copy(pk_hbm, pk_v)
    pltpu.sync_copy(val_hbm, vals_v)

    accs = ((acc00, acc01), (acc10, acc11))
    xbs = ((xb00, xb01), (xb10, xb11))
    xsems = (xsem0, xsem1)
    osems = (osem0, osem1)
    n_pairs = per_tile // 2

    pltpu.async_copy(x_hbm.at[b_base], xb00, xsem0)
    pltpu.async_copy(x_hbm.at[b_base + 1], xb01, xsem0)

    @plsc.parallel_loop(0, MG, unroll=8)
    def _zero0(k):
        s = pl.ds(k * L, L)
        z = jnp.zeros((L,), jnp.float32)
        acc00[s] = z
        acc01[s] = z

    HALF = EG // 2

    for p in range(n_pairs):
        par = p & 1
        a0, a1 = accs[par]
        o0, o1 = accs[1 - par]
        x0, x1 = xbs[par]
        b0 = b_base + 2 * p

        if p + 1 < n_pairs:
            nxt = b_base + 2 * (p + 1)
            pltpu.async_copy(x_hbm.at[nxt], xbs[1 - par][0], xsems[1 - par])
            pltpu.async_copy(x_hbm.at[nxt + 1], xbs[1 - par][1], xsems[1 - par])

        pltpu.make_async_copy(x_hbm.at[b0], x0, xsems[par]).wait()
        pltpu.make_async_copy(x_hbm.at[b0 + 1], x1, xsems[par]).wait()

        @plsc.parallel_loop(0, HALF, unroll=4)
        def _entry_a(g):
            s = pl.ds(g * L, L)
            pk = pk_v[s]
            v = vals_v[s]
            r = pk >> 12
            c = pk & 4095
            plsc.addupdate_scatter(a0, [r], plsc.load_gather(x0, [c]) * v)
            plsc.addupdate_scatter(a1, [r], plsc.load_gather(x1, [c]) * v)

        if p >= 1:
            prv = b_base + 2 * (p - 1)
            pltpu.make_async_copy(o0, out_hbm.at[prv], osems[1 - par]).wait()
            pltpu.make_async_copy(o1, out_hbm.at[prv + 1], osems[1 - par]).wait()

        @plsc.parallel_loop(HALF, EG, unroll=4)
        def _entry_b(g):
            s = pl.ds(g * L, L)
            pk = pk_v[s]
            v = vals_v[s]
            r = pk >> 12
            c = pk & 4095
            plsc.addupdate_scatter(a0, [r], plsc.load_gather(x0, [c]) * v)
            plsc.addupdate_scatter(a1, [r], plsc.load_gather(x1, [c]) * v)
            z = jnp.zeros((L,), jnp.float32)
            k = (g - HALF) * 2
            o0[pl.ds(k * L, L)] = z
            o0[pl.ds((k + 1) * L, L)] = z
            o1[pl.ds(k * L, L)] = z
            o1[pl.ds((k + 1) * L, L)] = z

        pltpu.async_copy(a0, out_hbm.at[b0], osems[par])
        pltpu.async_copy(a1, out_hbm.at[b0 + 1], osems[par])

    p = n_pairs - 1
    par = p & 1
    a0, a1 = accs[par]
    b0 = b_base + 2 * p
    pltpu.make_async_copy(a0, out_hbm.at[b0], osems[par]).wait()
    pltpu.make_async_copy(a1, out_hbm.at[b0 + 1], osems[par]).wait()


@jax.jit
def _sc_spmm(x, packed, vals):
    mesh = plsc.VectorSubcoreMesh(core_axis_name="c", subcore_axis_name="s")
    kfn = functools.partial(
        pl.kernel,
        out_type=jax.ShapeDtypeStruct((B, M), jnp.float32),
        mesh=mesh,
        compiler_params=pltpu.CompilerParams(needs_layout_passes=False),
        scratch_types=[
            pltpu.VMEM((E,), jnp.int32),
            pltpu.VMEM((E,), jnp.float32),
            pltpu.VMEM((M,), jnp.float32),
            pltpu.VMEM((M,), jnp.float32),
            pltpu.VMEM((M,), jnp.float32),
            pltpu.VMEM((M,), jnp.float32),
            pltpu.VMEM((N,), jnp.float32),
            pltpu.VMEM((N,), jnp.float32),
            pltpu.VMEM((N,), jnp.float32),
            pltpu.VMEM((N,), jnp.float32),
            pltpu.SemaphoreType.DMA,
            pltpu.SemaphoreType.DMA,
            pltpu.SemaphoreType.DMA,
            pltpu.SemaphoreType.DMA,
        ],
    )(_sc_body)
    return kfn(x, packed, vals)


def kernel(input, weight_indices, weight_values):
    wi = weight_indices.astype(jnp.int32)
    packed = wi[0] * 4096 + wi[1]
    return _sc_spmm(input, packed, weight_values)


# speedup vs baseline: 1.0877x; 1.0220x over previous
import functools

import jax
import jax.numpy as jnp
from jax import lax
from jax.experimental import pallas as pl
from jax.experimental.pallas import tpu as pltpu
from jax.experimental.pallas import tpu_sc as plsc

B = 1024
N = 4096
M = 16384
E = 16384
L = 16
EG = E // L
MG = M // L


def _sc_body(x_hbm, pk_hbm, val_hbm, out_hbm,
             pk_v, vals_v,
             acc00, acc01, acc10, acc11,
             xb00, xb01, xb10, xb11,
             w0, w1,
             xsem0, xsem1, osem0, osem1):
    info = plsc.get_sparse_core_info()
    nc = info.num_cores
    per_tile = B // (nc * info.num_subcores)
    wid = lax.axis_index("s") * nc + lax.axis_index("c")
    b_base = wid * per_tile

    pltpu.sync_copy(pk_hbm, pk_v)
    pltpu.sync_copy(val_hbm, vals_v)

    accs = ((acc00, acc01), (acc10, acc11))
    xbs = ((xb00, xb01), (xb10, xb11))
    ws = (w0, w1)
    xsems = (xsem0, xsem1)
    osems = (osem0, osem1)
    n_pairs = per_tile // 2

    pltpu.async_copy(x_hbm.at[b_base], xb00, xsem0)
    pltpu.async_copy(x_hbm.at[b_base + 1], xb01, xsem0)

    @plsc.parallel_loop(0, MG, unroll=8)
    def _zero0(k):
        s = pl.ds(k * L, L)
        z = jnp.zeros((L,), jnp.float32)
        acc00[s] = z
        acc01[s] = z

    HALF = EG // 2

    for p in range(n_pairs):
        par = p & 1
        a0, a1 = accs[par]
        o0, o1 = accs[1 - par]
        x0, x1 = xbs[par]
        b0 = b_base + 2 * p

        if p + 1 < n_pairs:
            nxt = b_base + 2 * (p + 1)
            pltpu.async_copy(x_hbm.at[nxt], xbs[1 - par][0], xsems[1 - par])
            pltpu.async_copy(x_hbm.at[nxt + 1], xbs[1 - par][1], xsems[1 - par])

        pltpu.make_async_copy(x_hbm.at[b0], x0, xsems[par]).wait()
        pltpu.make_async_copy(x_hbm.at[b0 + 1], x1, xsems[par]).wait()

        w = ws[par]

        @plsc.parallel_loop(0, N // L, unroll=4)
        def _packx(k):
            s = pl.ds(k * L, L)
            ab = plsc.pack(x0[s], x1[s], format=plsc.PackFormat.INTERLEAVED)
            w[s] = plsc.bitcast(ab, jnp.int32)

        @plsc.parallel_loop(0, HALF, unroll=4)
        def _entry_a(g):
            s = pl.ds(g * L, L)
            pk = pk_v[s]
            v = vals_v[s]
            r = pk >> 12
            c = pk & 4095
            ab = plsc.bitcast(plsc.load_gather(w, [c]), jnp.bfloat16)
            xv0, xv1 = plsc.unpack(ab, format=plsc.PackFormat.INTERLEAVED,
                                   preferred_element_type=jnp.float32)
            plsc.addupdate_scatter(a0, [r], xv0 * v)
            plsc.addupdate_scatter(a1, [r], xv1 * v)

        if p >= 1:
            prv = b_base + 2 * (p - 1)
            pltpu.make_async_copy(o0, out_hbm.at[prv], osems[1 - par]).wait()
            pltpu.make_async_copy(o1, out_hbm.at[prv + 1], osems[1 - par]).wait()

        @plsc.parallel_loop(HALF, EG, unroll=4)
        def _entry_b(g):
            s = pl.ds(g * L, L)
            pk = pk_v[s]
            v = vals_v[s]
            r = pk >> 12
            c = pk & 4095
            ab = plsc.bitcast(plsc.load_gather(w, [c]), jnp.bfloat16)
            xv0, xv1 = plsc.unpack(ab, format=plsc.PackFormat.INTERLEAVED,
                                   preferred_element_type=jnp.float32)
            plsc.addupdate_scatter(a0, [r], xv0 * v)
            plsc.addupdate_scatter(a1, [r], xv1 * v)
            z = jnp.zeros((L,), jnp.float32)
            k = (g - HALF) * 2
            o0[pl.ds(k * L, L)] = z
            o0[pl.ds((k + 1) * L, L)] = z
            o1[pl.ds(k * L, L)] = z
            o1[pl.ds((k + 1) * L, L)] = z

        pltpu.async_copy(a0, out_hbm.at[b0], osems[par])
        pltpu.async_copy(a1, out_hbm.at[b0 + 1], osems[par])

    p = n_pairs - 1
    par = p & 1
    a0, a1 = accs[par]
    b0 = b_base + 2 * p
    pltpu.make_async_copy(a0, out_hbm.at[b0], osems[par]).wait()
    pltpu.make_async_copy(a1, out_hbm.at[b0 + 1], osems[par]).wait()


@jax.jit
def _sc_spmm(x, packed, vals):
    mesh = plsc.VectorSubcoreMesh(core_axis_name="c", subcore_axis_name="s")
    kfn = functools.partial(
        pl.kernel,
        out_type=jax.ShapeDtypeStruct((B, M), jnp.float32),
        mesh=mesh,
        compiler_params=pltpu.CompilerParams(needs_layout_passes=False),
        scratch_types=[
            pltpu.VMEM((E,), jnp.int32),
            pltpu.VMEM((E,), jnp.float32),
            pltpu.VMEM((M,), jnp.float32),
            pltpu.VMEM((M,), jnp.float32),
            pltpu.VMEM((M,), jnp.float32),
            pltpu.VMEM((M,), jnp.float32),
            pltpu.VMEM((N,), jnp.float32),
            pltpu.VMEM((N,), jnp.float32),
            pltpu.VMEM((N,), jnp.float32),
            pltpu.VMEM((N,), jnp.float32),
            pltpu.VMEM((N,), jnp.int32),
            pltpu.VMEM((N,), jnp.int32),
            pltpu.SemaphoreType.DMA,
            pltpu.SemaphoreType.DMA,
            pltpu.SemaphoreType.DMA,
            pltpu.SemaphoreType.DMA,
        ],
    )(_sc_body)
    return kfn(x, packed, vals)


def kernel(input, weight_indices, weight_values):
    wi = weight_indices.astype(jnp.int32)
    packed = wi[0] * 4096 + wi[1]
    return _sc_spmm(input, packed, weight_values)


# speedup vs baseline: 1.1229x; 1.0323x over previous
import functools

import jax
import jax.numpy as jnp
from jax import lax
from jax.experimental import pallas as pl
from jax.experimental.pallas import tpu as pltpu
from jax.experimental.pallas import tpu_sc as plsc

B = 1024
N = 4096
M = 16384
E = 16384
L = 16
EG = E // L
MG = M // L


def _sc_body(x_hbm, pk_hbm, val_hbm, out_hbm,
             pk_v, vals_v,
             acc00, acc01, acc10, acc11,
             xb00, xb01, xb10, xb11,
             w0, w1, zshared,
             xsem0, xsem1, osem0, osem1, zsem0, zsem1):
    info = plsc.get_sparse_core_info()
    nc = info.num_cores
    per_tile = B // (nc * info.num_subcores)
    wid = lax.axis_index("s") * nc + lax.axis_index("c")
    b_base = wid * per_tile

    pltpu.sync_copy(pk_hbm, pk_v)
    pltpu.sync_copy(val_hbm, vals_v)

    accs = ((acc00, acc01), (acc10, acc11))
    xbs = ((xb00, xb01), (xb10, xb11))
    ws = (w0, w1)
    xsems = (xsem0, xsem1)
    osems = (osem0, osem1)
    zsems = (zsem0, zsem1)
    n_pairs = per_tile // 2

    pltpu.async_copy(x_hbm.at[b_base], xb00, xsem0)
    pltpu.async_copy(x_hbm.at[b_base + 1], xb01, xsem0)

    @plsc.parallel_loop(0, MG, unroll=8)
    def _zero0(k):
        s = pl.ds(k * L, L)
        z = jnp.zeros((L,), jnp.float32)
        acc00[s] = z
        acc01[s] = z

    sid = lax.axis_index("s")
    seg = M // info.num_subcores
    pltpu.sync_copy(acc00.at[pl.ds(0, seg)], zshared.at[pl.ds(sid * seg, seg)])
    plsc.subcore_barrier()

    HALF = EG // 2

    for p in range(n_pairs):
        par = p & 1
        a0, a1 = accs[par]
        o0, o1 = accs[1 - par]
        x0, x1 = xbs[par]
        b0 = b_base + 2 * p

        if p + 1 < n_pairs:
            nxt = b_base + 2 * (p + 1)
            pltpu.async_copy(x_hbm.at[nxt], xbs[1 - par][0], xsems[1 - par])
            pltpu.async_copy(x_hbm.at[nxt + 1], xbs[1 - par][1], xsems[1 - par])

        if p >= 1:
            pltpu.make_async_copy(zshared, a0, zsems[par]).wait()
            pltpu.make_async_copy(zshared, a1, zsems[par]).wait()

        pltpu.make_async_copy(x_hbm.at[b0], x0, xsems[par]).wait()
        pltpu.make_async_copy(x_hbm.at[b0 + 1], x1, xsems[par]).wait()

        w = ws[par]

        @plsc.parallel_loop(0, N // L, unroll=4)
        def _packx(k):
            s = pl.ds(k * L, L)
            ab = plsc.pack(x0[s], x1[s], format=plsc.PackFormat.INTERLEAVED)
            w[s] = plsc.bitcast(ab, jnp.int32)

        @plsc.parallel_loop(0, HALF, unroll=4)
        def _entry_a(g):
            s = pl.ds(g * L, L)
            pk = pk_v[s]
            v = vals_v[s]
            r = pk >> 12
            c = pk & 4095
            ab = plsc.bitcast(plsc.load_gather(w, [c]), jnp.bfloat16)
            xv0, xv1 = plsc.unpack(ab, format=plsc.PackFormat.INTERLEAVED,
                                   preferred_element_type=jnp.float32)
            plsc.addupdate_scatter(a0, [r], xv0 * v)
            plsc.addupdate_scatter(a1, [r], xv1 * v)

        if p >= 1:
            prv = b_base + 2 * (p - 1)
            pltpu.make_async_copy(o0, out_hbm.at[prv], osems[1 - par]).wait()
            pltpu.make_async_copy(o1, out_hbm.at[prv + 1], osems[1 - par]).wait()
        if p + 1 < n_pairs:
            pltpu.async_copy(zshared, o0, zsems[1 - par])
            pltpu.async_copy(zshared, o1, zsems[1 - par])

        @plsc.parallel_loop(HALF, EG, unroll=4)
        def _entry_b(g):
            s = pl.ds(g * L, L)
            pk = pk_v[s]
            v = vals_v[s]
            r = pk >> 12
            c = pk & 4095
            ab = plsc.bitcast(plsc.load_gather(w, [c]), jnp.bfloat16)
            xv0, xv1 = plsc.unpack(ab, format=plsc.PackFormat.INTERLEAVED,
                                   preferred_element_type=jnp.float32)
            plsc.addupdate_scatter(a0, [r], xv0 * v)
            plsc.addupdate_scatter(a1, [r], xv1 * v)

        pltpu.async_copy(a0, out_hbm.at[b0], osems[par])
        pltpu.async_copy(a1, out_hbm.at[b0 + 1], osems[par])

    p = n_pairs - 1
    par = p & 1
    a0, a1 = accs[par]
    b0 = b_base + 2 * p
    pltpu.make_async_copy(a0, out_hbm.at[b0], osems[par]).wait()
    pltpu.make_async_copy(a1, out_hbm.at[b0 + 1], osems[par]).wait()


@jax.jit
def _sc_spmm(x, packed, vals):
    mesh = plsc.VectorSubcoreMesh(core_axis_name="c", subcore_axis_name="s")
    kfn = functools.partial(
        pl.kernel,
        out_type=jax.ShapeDtypeStruct((B, M), jnp.float32),
        mesh=mesh,
        compiler_params=pltpu.CompilerParams(needs_layout_passes=False),
        scratch_types=[
            pltpu.VMEM((E,), jnp.int32),
            pltpu.VMEM((E,), jnp.float32),
            pltpu.VMEM((M,), jnp.float32),
            pltpu.VMEM((M,), jnp.float32),
            pltpu.VMEM((M,), jnp.float32),
            pltpu.VMEM((M,), jnp.float32),
            pltpu.VMEM((N,), jnp.float32),
            pltpu.VMEM((N,), jnp.float32),
            pltpu.VMEM((N,), jnp.float32),
            pltpu.VMEM((N,), jnp.float32),
            pltpu.VMEM((N,), jnp.int32),
            pltpu.VMEM((N,), jnp.int32),
            pltpu.VMEM_SHARED((M,), jnp.float32),
            pltpu.SemaphoreType.DMA,
            pltpu.SemaphoreType.DMA,
            pltpu.SemaphoreType.DMA,
            pltpu.SemaphoreType.DMA,
            pltpu.SemaphoreType.DMA,
            pltpu.SemaphoreType.DMA,
        ],
    )(_sc_body)
    return kfn(x, packed, vals)


def kernel(input, weight_indices, weight_values):
    wi = weight_indices.astype(jnp.int32)
    packed = wi[0] * 4096 + wi[1]
    return _sc_spmm(input, packed, weight_values)
